# baseline (speedup 1.0000x reference)
import functools

import jax
import jax.numpy as jnp
from jax import lax
from jax.experimental import pallas as pl
from jax.experimental.pallas import tpu as pltpu
from jax.experimental.pallas import tpu_sc as plsc

N_ENT = 1000000
N_REL = 1000000
IN_DIM = 128
HALF = IN_DIM // 2
BATCH = 16384
MARGIN = 6.0
EPSILON = 2.0
EMB_RANGE = (MARGIN + EPSILON) / IN_DIM
PI = 3.141592653589793
PHASE_SCALE = PI / EMB_RANGE

NC = 2
NS = 16
NW = NC * NS
B_PER_W = BATCH // NW
CHUNK = 128
N_CHUNKS = B_PER_W // CHUNK

RW = 2048
RGRID = (N_REL + 2 * RW - 1) // (2 * RW)
NPAIR = RGRID * RW


def _relayout_body(x_ref, out_ref):
    x = x_ref[...]
    eye = jnp.eye(HALF, dtype=jnp.float32)
    t1 = lax.dot_general(x[:, :RW], eye, (((0,), (0,)), ((), ())),
                         precision=lax.Precision.HIGHEST,
                         preferred_element_type=jnp.float32)
    t2 = lax.dot_general(x[:, RW:], eye, (((0,), (0,)), ((), ())),
                         precision=lax.Precision.HIGHEST,
                         preferred_element_type=jnp.float32)
    out_ref[...] = jnp.concatenate([t1, t2], axis=1)


def _tc_relayout(rel_t):
    return pl.pallas_call(
        _relayout_body,
        grid=(RGRID,),
        in_specs=[pl.BlockSpec((HALF, 2 * RW), lambda i: (0, i))],
        out_specs=pl.BlockSpec((RW, IN_DIM), lambda i: (i, 0)),
        out_shape=jax.ShapeDtypeStruct((NPAIR, IN_DIM), jnp.float32),
    )(rel_t)


def _sc_gather(h_idx, g_idx, t_idx, ent_embed, rel_pair):
    mesh = plsc.VectorSubcoreMesh(core_axis_name="c", subcore_axis_name="s")

    @functools.partial(
        pl.kernel,
        mesh=mesh,
        out_type=(
            jax.ShapeDtypeStruct((BATCH, IN_DIM), jnp.float32),
            jax.ShapeDtypeStruct((BATCH, IN_DIM), jnp.float32),
            jax.ShapeDtypeStruct((BATCH, IN_DIM), jnp.float32),
        ),
        scratch_types=(
            pltpu.VMEM((CHUNK,), jnp.int32),
            pltpu.VMEM((CHUNK,), jnp.int32),
            pltpu.VMEM((CHUNK,), jnp.int32),
            pltpu.VMEM((CHUNK, IN_DIM), jnp.float32),
            pltpu.VMEM((CHUNK, IN_DIM), jnp.float32),
            pltpu.VMEM((CHUNK, IN_DIM), jnp.float32),
            pltpu.SemaphoreType.DMA,
        ),
    )
    def k(h_idx_hbm, g_idx_hbm, t_idx_hbm, ent_hbm, relp_hbm,
          head_out, tail_out, relp_out,
          hidx_v, gidx_v, tidx_v, head_v, tail_v, relp_v, sem):
        wid = lax.axis_index("s") * NC + lax.axis_index("c")
        base = wid * B_PER_W
        for c in range(N_CHUNKS):
            off = base + c * CHUNK
            pltpu.sync_copy(h_idx_hbm.at[pl.ds(off, CHUNK)], hidx_v)
            pltpu.sync_copy(t_idx_hbm.at[pl.ds(off, CHUNK)], tidx_v)
            pltpu.sync_copy(g_idx_hbm.at[pl.ds(off, CHUNK)], gidx_v)
            cp_h = pltpu.async_copy(ent_hbm.at[hidx_v], head_v, sem)
            cp_t = pltpu.async_copy(ent_hbm.at[tidx_v], tail_v, sem)
            cp_r = pltpu.async_copy(relp_hbm.at[gidx_v], relp_v, sem)
            cp_h.wait()
            cp_t.wait()
            cp_r.wait()
            pltpu.sync_copy(head_v, head_out.at[pl.ds(off, CHUNK)])
            pltpu.sync_copy(tail_v, tail_out.at[pl.ds(off, CHUNK)])
            pltpu.sync_copy(relp_v, relp_out.at[pl.ds(off, CHUNK)])

    return k(h_idx, g_idx, t_idx, ent_embed, rel_pair)


def _tc_math_body(head_ref, tail_ref, relp_ref, par_ref, out_ref):
    head = head_ref[...]
    tail = tail_ref[...]
    relp = relp_ref[...]
    par = par_ref[...]
    rel = jnp.where(par == 0, relp[:, :HALF], relp[:, HALF:])
    phase = rel * PHASE_SCALE
    re_r = jnp.cos(phase)
    im_r = jnp.sin(phase)
    re_h = head[:, :HALF]
    im_h = head[:, HALF:]
    re_s = re_h * re_r - im_h * im_r - tail[:, :HALF]
    im_s = re_h * im_r + im_h * re_r - tail[:, HALF:]
    s = jnp.sqrt(re_s * re_s + im_s * im_s)
    out_ref[...] = MARGIN - jnp.sum(s, axis=1, keepdims=True)


def _tc_math(head, tail, relp, par):
    blk = 2048
    grid = (BATCH // blk,)
    return pl.pallas_call(
        _tc_math_body,
        grid=grid,
        in_specs=[
            pl.BlockSpec((blk, IN_DIM), lambda i: (i, 0)),
            pl.BlockSpec((blk, IN_DIM), lambda i: (i, 0)),
            pl.BlockSpec((blk, IN_DIM), lambda i: (i, 0)),
            pl.BlockSpec((blk, 1), lambda i: (i, 0)),
        ],
        out_specs=pl.BlockSpec((blk, 1), lambda i: (i, 0)),
        out_shape=jax.ShapeDtypeStruct((BATCH, 1), jnp.float32),
    )(head, tail, relp, par)


def kernel(sample, ent_embed, rel_embed):
    h_idx = sample[:, 0]
    r_idx = sample[:, 1]
    t_idx = sample[:, 2]
    loc = r_idx % (2 * RW)
    g_idx = (r_idx // (2 * RW)) * RW + (loc % RW)
    par = (loc // RW)[:, None]
    rel_pair = _tc_relayout(rel_embed.T)
    head, tail, relp = _sc_gather(h_idx, g_idx, t_idx, ent_embed, rel_pair)
    return _tc_math(head, tail, relp, par)


# speedup vs baseline: 1.4728x; 1.4728x over previous
import functools

import jax
import jax.numpy as jnp
from jax import lax
from jax.experimental import pallas as pl
from jax.experimental.pallas import tpu as pltpu
from jax.experimental.pallas import tpu_sc as plsc

N_ENT = 1000000
N_REL = 1000000
IN_DIM = 128
HALF = IN_DIM // 2
BATCH = 16384
MARGIN = 6.0
EPSILON = 2.0
EMB_RANGE = (MARGIN + EPSILON) / IN_DIM
PI = 3.141592653589793
PHASE_SCALE = PI / EMB_RANGE

NC = 2
NS = 16
NW = NC * NS
B_PER_W = BATCH // NW
CHUNK = 128
N_CHUNKS = B_PER_W // CHUNK

RW = 2048
RGRID = (N_REL + 2 * RW - 1) // (2 * RW)
NPAIR = RGRID * RW


def _relayout_body(x_ref, out_ref):
    x = x_ref[...]
    t1 = x[:, :RW].T
    t2 = x[:, RW:].T
    out_ref[...] = jnp.concatenate([t1, t2], axis=1)


def _tc_relayout(rel_t):
    return pl.pallas_call(
        _relayout_body,
        grid=(RGRID,),
        in_specs=[pl.BlockSpec((HALF, 2 * RW), lambda i: (0, i))],
        out_specs=pl.BlockSpec((RW, IN_DIM), lambda i: (i, 0)),
        out_shape=jax.ShapeDtypeStruct((NPAIR, IN_DIM), jnp.float32),
    )(rel_t)


def _sc_gather(h_idx, g_idx, t_idx, ent_embed, rel_pair):
    mesh = plsc.VectorSubcoreMesh(core_axis_name="c", subcore_axis_name="s")

    @functools.partial(
        pl.kernel,
        mesh=mesh,
        out_type=(
            jax.ShapeDtypeStruct((BATCH, IN_DIM), jnp.float32),
            jax.ShapeDtypeStruct((BATCH, IN_DIM), jnp.float32),
            jax.ShapeDtypeStruct((BATCH, IN_DIM), jnp.float32),
        ),
        scratch_types=(
            pltpu.VMEM((CHUNK,), jnp.int32),
            pltpu.VMEM((CHUNK,), jnp.int32),
            pltpu.VMEM((CHUNK,), jnp.int32),
            pltpu.VMEM((CHUNK, IN_DIM), jnp.float32),
            pltpu.VMEM((CHUNK, IN_DIM), jnp.float32),
            pltpu.VMEM((CHUNK, IN_DIM), jnp.float32),
            pltpu.SemaphoreType.DMA,
        ),
    )
    def k(h_idx_hbm, g_idx_hbm, t_idx_hbm, ent_hbm, relp_hbm,
          head_out, tail_out, relp_out,
          hidx_v, gidx_v, tidx_v, head_v, tail_v, relp_v, sem):
        wid = lax.axis_index("s") * NC + lax.axis_index("c")
        base = wid * B_PER_W
        for c in range(N_CHUNKS):
            off = base + c * CHUNK
            pltpu.sync_copy(h_idx_hbm.at[pl.ds(off, CHUNK)], hidx_v)
            pltpu.sync_copy(t_idx_hbm.at[pl.ds(off, CHUNK)], tidx_v)
            pltpu.sync_copy(g_idx_hbm.at[pl.ds(off, CHUNK)], gidx_v)
            cp_h = pltpu.async_copy(ent_hbm.at[hidx_v], head_v, sem)
            cp_t = pltpu.async_copy(ent_hbm.at[tidx_v], tail_v, sem)
            cp_r = pltpu.async_copy(relp_hbm.at[gidx_v], relp_v, sem)
            cp_h.wait()
            cp_t.wait()
            cp_r.wait()
            pltpu.sync_copy(head_v, head_out.at[pl.ds(off, CHUNK)])
            pltpu.sync_copy(tail_v, tail_out.at[pl.ds(off, CHUNK)])
            pltpu.sync_copy(relp_v, relp_out.at[pl.ds(off, CHUNK)])

    return k(h_idx, g_idx, t_idx, ent_embed, rel_pair)


def _tc_math_body(head_ref, tail_ref, relp_ref, par_ref, out_ref):
    head = head_ref[...]
    tail = tail_ref[...]
    relp = relp_ref[...]
    par = par_ref[...]
    rel = jnp.where(par == 0, relp[:, :HALF], relp[:, HALF:])
    phase = rel * PHASE_SCALE
    re_r = jnp.cos(phase)
    im_r = jnp.sin(phase)
    re_h = head[:, :HALF]
    im_h = head[:, HALF:]
    re_s = re_h * re_r - im_h * im_r - tail[:, :HALF]
    im_s = re_h * im_r + im_h * re_r - tail[:, HALF:]
    s = jnp.sqrt(re_s * re_s + im_s * im_s)
    out_ref[...] = MARGIN - jnp.sum(s, axis=1, keepdims=True)


def _tc_math(head, tail, relp, par):
    blk = 2048
    grid = (BATCH // blk,)
    return pl.pallas_call(
        _tc_math_body,
        grid=grid,
        in_specs=[
            pl.BlockSpec((blk, IN_DIM), lambda i: (i, 0)),
            pl.BlockSpec((blk, IN_DIM), lambda i: (i, 0)),
            pl.BlockSpec((blk, IN_DIM), lambda i: (i, 0)),
            pl.BlockSpec((blk, 1), lambda i: (i, 0)),
        ],
        out_specs=pl.BlockSpec((blk, 1), lambda i: (i, 0)),
        out_shape=jax.ShapeDtypeStruct((BATCH, 1), jnp.float32),
    )(head, tail, relp, par)


def kernel(sample, ent_embed, rel_embed):
    h_idx = sample[:, 0]
    r_idx = sample[:, 1]
    t_idx = sample[:, 2]
    loc = r_idx % (2 * RW)
    g_idx = (r_idx // (2 * RW)) * RW + (loc % RW)
    par = (loc // RW)[:, None]
    rel_pair = _tc_relayout(rel_embed.T)
    head, tail, relp = _sc_gather(h_idx, g_idx, t_idx, ent_embed, rel_pair)
    return _tc_math(head, tail, relp, par)


# speedup vs baseline: 1.7473x; 1.1864x over previous
import functools

import jax
import jax.numpy as jnp
from jax import lax
from jax.experimental import pallas as pl
from jax.experimental.pallas import tpu as pltpu
from jax.experimental.pallas import tpu_sc as plsc

N_ENT = 1000000
N_REL = 1000000
IN_DIM = 128
HALF = IN_DIM // 2
BATCH = 16384
MARGIN = 6.0
EPSILON = 2.0
EMB_RANGE = (MARGIN + EPSILON) / IN_DIM
PI = 3.141592653589793
PHASE_SCALE = PI / EMB_RANGE

NC = 2
NS = 16
NW = NC * NS
B_PER_W = BATCH // NW
CHUNK = 128
N_CHUNKS = B_PER_W // CHUNK

RW = 4096
RGRID = (N_REL + 2 * RW - 1) // (2 * RW)
NPAIR = RGRID * RW


def _relayout_body(x_ref, out_ref):
    x = x_ref[...]
    out_ref[:, :HALF] = x[:, :RW].T
    out_ref[:, HALF:] = x[:, RW:].T


def _tc_relayout(rel_t):
    return pl.pallas_call(
        _relayout_body,
        grid=(RGRID,),
        in_specs=[pl.BlockSpec((HALF, 2 * RW), lambda i: (0, i))],
        out_specs=pl.BlockSpec((RW, IN_DIM), lambda i: (i, 0)),
        out_shape=jax.ShapeDtypeStruct((NPAIR, IN_DIM), jnp.float32),
    )(rel_t)


def _sc_gather(h_idx, g_idx, t_idx, ent_embed, rel_pair):
    mesh = plsc.VectorSubcoreMesh(core_axis_name="c", subcore_axis_name="s")

    @functools.partial(
        pl.kernel,
        mesh=mesh,
        out_type=(
            jax.ShapeDtypeStruct((BATCH, IN_DIM), jnp.float32),
            jax.ShapeDtypeStruct((BATCH, IN_DIM), jnp.float32),
            jax.ShapeDtypeStruct((BATCH, IN_DIM), jnp.float32),
        ),
        scratch_types=(
            pltpu.VMEM((CHUNK,), jnp.int32),
            pltpu.VMEM((CHUNK,), jnp.int32),
            pltpu.VMEM((CHUNK,), jnp.int32),
            pltpu.VMEM((CHUNK, IN_DIM), jnp.float32),
            pltpu.VMEM((CHUNK, IN_DIM), jnp.float32),
            pltpu.VMEM((CHUNK, IN_DIM), jnp.float32),
            pltpu.SemaphoreType.DMA,
        ),
    )
    def k(h_idx_hbm, g_idx_hbm, t_idx_hbm, ent_hbm, relp_hbm,
          head_out, tail_out, relp_out,
          hidx_v, gidx_v, tidx_v, head_v, tail_v, relp_v, sem):
        wid = lax.axis_index("s") * NC + lax.axis_index("c")
        base = wid * B_PER_W
        for c in range(N_CHUNKS):
            off = base + c * CHUNK
            pltpu.sync_copy(h_idx_hbm.at[pl.ds(off, CHUNK)], hidx_v)
            pltpu.sync_copy(t_idx_hbm.at[pl.ds(off, CHUNK)], tidx_v)
            pltpu.sync_copy(g_idx_hbm.at[pl.ds(off, CHUNK)], gidx_v)
            cp_h = pltpu.async_copy(ent_hbm.at[hidx_v], head_v, sem)
            cp_t = pltpu.async_copy(ent_hbm.at[tidx_v], tail_v, sem)
            cp_r = pltpu.async_copy(relp_hbm.at[gidx_v], relp_v, sem)
            cp_h.wait()
            cp_t.wait()
            cp_r.wait()
            pltpu.sync_copy(head_v, head_out.at[pl.ds(off, CHUNK)])
            pltpu.sync_copy(tail_v, tail_out.at[pl.ds(off, CHUNK)])
            pltpu.sync_copy(relp_v, relp_out.at[pl.ds(off, CHUNK)])

    return k(h_idx, g_idx, t_idx, ent_embed, rel_pair)


def _tc_math_body(head_ref, tail_ref, relp_ref, par_ref, out_ref):
    head = head_ref[...]
    tail = tail_ref[...]
    relp = relp_ref[...]
    par = par_ref[...]
    rel = jnp.where(par == 0, relp[:, :HALF], relp[:, HALF:])
    phase = rel * PHASE_SCALE
    re_r = jnp.cos(phase)
    im_r = jnp.sin(phase)
    re_h = head[:, :HALF]
    im_h = head[:, HALF:]
    re_s = re_h * re_r - im_h * im_r - tail[:, :HALF]
    im_s = re_h * im_r + im_h * re_r - tail[:, HALF:]
    s = jnp.sqrt(re_s * re_s + im_s * im_s)
    out_ref[...] = MARGIN - jnp.sum(s, axis=1, keepdims=True)


def _tc_math(head, tail, relp, par):
    blk = 2048
    grid = (BATCH // blk,)
    return pl.pallas_call(
        _tc_math_body,
        grid=grid,
        in_specs=[
            pl.BlockSpec((blk, IN_DIM), lambda i: (i, 0)),
            pl.BlockSpec((blk, IN_DIM), lambda i: (i, 0)),
            pl.BlockSpec((blk, IN_DIM), lambda i: (i, 0)),
            pl.BlockSpec((blk, 1), lambda i: (i, 0)),
        ],
        out_specs=pl.BlockSpec((blk, 1), lambda i: (i, 0)),
        out_shape=jax.ShapeDtypeStruct((BATCH, 1), jnp.float32),
    )(head, tail, relp, par)


def kernel(sample, ent_embed, rel_embed):
    h_idx = sample[:, 0]
    r_idx = sample[:, 1]
    t_idx = sample[:, 2]
    loc = r_idx % (2 * RW)
    g_idx = (r_idx // (2 * RW)) * RW + (loc % RW)
    par = (loc // RW)[:, None]
    rel_pair = _tc_relayout(rel_embed.T)
    head, tail, relp = _sc_gather(h_idx, g_idx, t_idx, ent_embed, rel_pair)
    return _tc_math(head, tail, relp, par)


# speedup vs baseline: 1.9180x; 1.0977x over previous
import functools

import jax
import jax.numpy as jnp
from jax import lax
from jax.experimental import pallas as pl
from jax.experimental.pallas import tpu as pltpu
from jax.experimental.pallas import tpu_sc as plsc

N_ENT = 1000000
N_REL = 1000000
IN_DIM = 128
HALF = IN_DIM // 2
BATCH = 16384
MARGIN = 6.0
EPSILON = 2.0
EMB_RANGE = (MARGIN + EPSILON) / IN_DIM
PI = 3.141592653589793
PHASE_SCALE = PI / EMB_RANGE

NC = 2
NS = 16
NW = NC * NS
B_PER_W = BATCH // NW
CHUNK = 128
N_CHUNKS = B_PER_W // CHUNK

RW = 8192
RGRID = (N_REL + 2 * RW - 1) // (2 * RW)
NPAIR = RGRID * RW


def _relayout_body(x_ref, out_ref):
    x = x_ref[...]
    out_ref[:, :HALF] = x[:, :RW].T
    out_ref[:, HALF:] = x[:, RW:].T


def _tc_relayout(rel_t):
    return pl.pallas_call(
        _relayout_body,
        grid=(RGRID,),
        in_specs=[pl.BlockSpec((HALF, 2 * RW), lambda i: (0, i))],
        out_specs=pl.BlockSpec((RW, IN_DIM), lambda i: (i, 0)),
        out_shape=jax.ShapeDtypeStruct((NPAIR, IN_DIM), jnp.float32),
    )(rel_t)


def _sc_gather(h_idx, g_idx, t_idx, ent_embed, rel_pair):
    mesh = plsc.VectorSubcoreMesh(core_axis_name="c", subcore_axis_name="s")

    @functools.partial(
        pl.kernel,
        mesh=mesh,
        out_type=(
            jax.ShapeDtypeStruct((BATCH, IN_DIM), jnp.float32),
            jax.ShapeDtypeStruct((BATCH, IN_DIM), jnp.float32),
            jax.ShapeDtypeStruct((BATCH, IN_DIM), jnp.float32),
        ),
        scratch_types=(
            pltpu.VMEM((CHUNK,), jnp.int32),
            pltpu.VMEM((CHUNK,), jnp.int32),
            pltpu.VMEM((CHUNK,), jnp.int32),
            pltpu.VMEM((CHUNK, IN_DIM), jnp.float32),
            pltpu.VMEM((CHUNK, IN_DIM), jnp.float32),
            pltpu.VMEM((CHUNK, IN_DIM), jnp.float32),
            pltpu.SemaphoreType.DMA,
        ),
    )
    def k(h_idx_hbm, g_idx_hbm, t_idx_hbm, ent_hbm, relp_hbm,
          head_out, tail_out, relp_out,
          hidx_v, gidx_v, tidx_v, head_v, tail_v, relp_v, sem):
        wid = lax.axis_index("s") * NC + lax.axis_index("c")
        base = wid * B_PER_W
        for c in range(N_CHUNKS):
            off = base + c * CHUNK
            pltpu.sync_copy(h_idx_hbm.at[pl.ds(off, CHUNK)], hidx_v)
            pltpu.sync_copy(t_idx_hbm.at[pl.ds(off, CHUNK)], tidx_v)
            pltpu.sync_copy(g_idx_hbm.at[pl.ds(off, CHUNK)], gidx_v)
            cp_h = pltpu.async_copy(ent_hbm.at[hidx_v], head_v, sem)
            cp_t = pltpu.async_copy(ent_hbm.at[tidx_v], tail_v, sem)
            cp_r = pltpu.async_copy(relp_hbm.at[gidx_v], relp_v, sem)
            cp_h.wait()
            cp_t.wait()
            cp_r.wait()
            pltpu.sync_copy(head_v, head_out.at[pl.ds(off, CHUNK)])
            pltpu.sync_copy(tail_v, tail_out.at[pl.ds(off, CHUNK)])
            pltpu.sync_copy(relp_v, relp_out.at[pl.ds(off, CHUNK)])

    return k(h_idx, g_idx, t_idx, ent_embed, rel_pair)


def _tc_math_body(head_ref, tail_ref, relp_ref, par_ref, out_ref):
    head = head_ref[...]
    tail = tail_ref[...]
    relp = relp_ref[...]
    par = par_ref[...]
    rel = jnp.where(par == 0, relp[:, :HALF], relp[:, HALF:])
    phase = rel * PHASE_SCALE
    re_r = jnp.cos(phase)
    im_r = jnp.sin(phase)
    re_h = head[:, :HALF]
    im_h = head[:, HALF:]
    re_s = re_h * re_r - im_h * im_r - tail[:, :HALF]
    im_s = re_h * im_r + im_h * re_r - tail[:, HALF:]
    s = jnp.sqrt(re_s * re_s + im_s * im_s)
    out_ref[...] = MARGIN - jnp.sum(s, axis=1, keepdims=True)


def _tc_math(head, tail, relp, par):
    blk = 2048
    grid = (BATCH // blk,)
    return pl.pallas_call(
        _tc_math_body,
        grid=grid,
        in_specs=[
            pl.BlockSpec((blk, IN_DIM), lambda i: (i, 0)),
            pl.BlockSpec((blk, IN_DIM), lambda i: (i, 0)),
            pl.BlockSpec((blk, IN_DIM), lambda i: (i, 0)),
            pl.BlockSpec((blk, 1), lambda i: (i, 0)),
        ],
        out_specs=pl.BlockSpec((blk, 1), lambda i: (i, 0)),
        out_shape=jax.ShapeDtypeStruct((BATCH, 1), jnp.float32),
    )(head, tail, relp, par)


def kernel(sample, ent_embed, rel_embed):
    h_idx = sample[:, 0]
    r_idx = sample[:, 1]
    t_idx = sample[:, 2]
    loc = r_idx % (2 * RW)
    g_idx = (r_idx // (2 * RW)) * RW + (loc % RW)
    par = (loc // RW)[:, None]
    rel_pair = _tc_relayout(rel_embed.T)
    head, tail, relp = _sc_gather(h_idx, g_idx, t_idx, ent_embed, rel_pair)
    return _tc_math(head, tail, relp, par)


# speedup vs baseline: 1.9343x; 1.0085x over previous
import functools

import jax
import jax.numpy as jnp
from jax import lax
from jax.experimental import pallas as pl
from jax.experimental.pallas import tpu as pltpu
from jax.experimental.pallas import tpu_sc as plsc

N_ENT = 1000000
N_REL = 1000000
IN_DIM = 128
HALF = IN_DIM // 2
BATCH = 16384
MARGIN = 6.0
EPSILON = 2.0
EMB_RANGE = (MARGIN + EPSILON) / IN_DIM
PI = 3.141592653589793
PHASE_SCALE = PI / EMB_RANGE

NC = 2
NS = 16
NW = NC * NS
B_PER_W = BATCH // NW
CHUNK = 128
N_CHUNKS = B_PER_W // CHUNK

SLAB = 256
NSLAB_TOT = (N_REL + SLAB - 1) // SLAB
SLAB_PW = 123
LISTCAP = BATCH + 16


def _sc_gather(h_idx, r_idx, t_idx, ent_embed, rel_t, tail_t):
    mesh = plsc.VectorSubcoreMesh(core_axis_name="c", subcore_axis_name="s")

    @functools.partial(
        pl.kernel,
        mesh=mesh,
        compiler_params=pltpu.CompilerParams(needs_layout_passes=False),
        out_type=(
            jax.ShapeDtypeStruct((BATCH, IN_DIM), jnp.float32),
            jax.ShapeDtypeStruct((BATCH, IN_DIM), jnp.float32),
            jax.ShapeDtypeStruct((BATCH * HALF,), jnp.float32),
        ),
        scratch_types=(
            pltpu.VMEM((CHUNK,), jnp.int32),
            pltpu.VMEM((CHUNK,), jnp.int32),
            pltpu.VMEM((CHUNK, IN_DIM), jnp.float32),
            pltpu.VMEM((CHUNK, IN_DIM), jnp.float32),
            pltpu.VMEM((BATCH,), jnp.int32),
            pltpu.VMEM((LISTCAP,), jnp.int32),
            pltpu.VMEM((LISTCAP,), jnp.int32),
            pltpu.VMEM((LISTCAP,), jnp.int32),
            pltpu.VMEM((HALF, SLAB), jnp.float32),
            pltpu.VMEM((HALF, 64), jnp.float32),
            pltpu.VMEM((8 * HALF,), jnp.float32),
            pltpu.SMEM((SLAB_PW + 8,), jnp.int32),
            pltpu.SMEM((SLAB_PW + 8,), jnp.int32),
            pltpu.SMEM((SLAB_PW + 8,), jnp.int32),
            pltpu.SemaphoreType.DMA,
            pltpu.SemaphoreType.DMA,
        ),
    )
    def k(h_idx_hbm, r_idx_hbm, t_idx_hbm, ent_hbm, relt_hbm, tailt_hbm,
          head_out, tail_out, rel_out,
          hidx_v, tidx_v, head_v, tail_v,
          ridx_all, lpack, lslab, lsort, slab_v, tbuf_v, stage_v,
          cnts_s, begs_s, curs_s, sem, osem):
        wid = lax.axis_index("s") * NC + lax.axis_index("c")
        base = wid * B_PER_W

        for c in range(N_CHUNKS):
            off = base + c * CHUNK
            pltpu.sync_copy(h_idx_hbm.at[pl.ds(off, CHUNK)], hidx_v)
            pltpu.sync_copy(t_idx_hbm.at[pl.ds(off, CHUNK)], tidx_v)
            cp_h = pltpu.async_copy(ent_hbm.at[hidx_v], head_v, sem)
            cp_t = pltpu.async_copy(ent_hbm.at[tidx_v], tail_v, sem)
            cp_h.wait()
            cp_t.wait()
            pltpu.sync_copy(head_v, head_out.at[pl.ds(off, CHUNK)])
            pltpu.sync_copy(tail_v, tail_out.at[pl.ds(off, CHUNK)])

        lo = wid * (SLAB_PW * SLAB)
        hi = jnp.minimum(lo + SLAB_PW * SLAB, N_REL)
        pltpu.sync_copy(r_idx_hbm, ridx_all)
        iota = lax.iota(jnp.int32, 16)
        lane0 = iota == 0

        def compress(g, cnt):
            v = ridx_all[pl.ds(g * 16, 16)]
            lo16 = jnp.full((16,), lo, jnp.int32)
            hi16 = jnp.full((16,), hi, jnp.int32)
            m = (v >= lo16) & (v < hi16)
            cs = plsc.cumsum(m.astype(jnp.int32))
            pos = jnp.full((16,), cnt, jnp.int32) + cs - 1
            rel_off = v - lo16
            b16 = iota + jnp.full((16,), g * 16, jnp.int32)
            pack = (b16 << 8) | (rel_off & (SLAB - 1))
            plsc.store_scatter(lpack, [pos], pack, mask=m)
            plsc.store_scatter(lslab, [pos], rel_off >> 8, mask=m)
            return cnt + cs[15]
        cnt = lax.fori_loop(0, BATCH // 16, compress, 0)

        def zero(i, _):
            cnts_s[i] = 0
            return 0
        lax.fori_loop(0, SLAB_PW + 1, zero, 0)

        def count(e, _):
            c = lslab[pl.ds(e, 16)][0]
            cnts_s[c] = cnts_s[c] + 1
            return 0
        lax.fori_loop(0, cnt, count, 0)

        def prefix(i, acc):
            begs_s[i] = acc
            curs_s[i] = acc
            return acc + cnts_s[i]
        total = lax.fori_loop(0, SLAB_PW + 1, prefix, 0)
        begs_s[SLAB_PW + 1] = total
        curs_s[SLAB_PW + 1] = total

        def scat(e, _):
            c = lslab[pl.ds(e, 16)][0]
            p = lpack[pl.ds(e, 16)][0]
            pos = curs_s[c]
            curs_s[c] = pos + 1
            plsc.store_scatter(lsort, [jnp.full((16,), pos, jnp.int32)],
                               jnp.full((16,), p, jnp.int32), mask=lane0)
            return 0
        lax.fori_loop(0, cnt, scat, 0)

        def extract_from(buf_ref, e, t):
            p = lsort[pl.ds(e, 16)][0]
            b = p >> 8
            col = p & (SLAB - 1)
            slot = t % 8

            @pl.when(t >= 8)
            def _():
                pltpu.make_async_copy(
                    rel_out.at[pl.ds(0, HALF)],
                    stage_v.at[pl.ds(0, HALF)], osem).wait()
            col16 = jnp.full((16,), col, jnp.int32)
            for kk in range(4):
                vals = plsc.load_gather(buf_ref, [iota + kk * 16, col16])
                stage_v[pl.ds(slot * HALF + kk * 16, 16)] = vals
            pltpu.async_copy(stage_v.at[pl.ds(slot * HALF, HALF)],
                             rel_out.at[pl.ds(b * HALF, HALF)], osem)
            return t + 1

        hi_full = jnp.minimum(lo + SLAB_PW * SLAB, (NSLAB_TOT - 1) * SLAB)
        nfull = (hi_full - lo) // SLAB

        def slab_loop(ch, t):
            pltpu.sync_copy(relt_hbm.at[:, pl.ds(lo + ch * SLAB, SLAB)],
                            slab_v)

            def extract(e, t):
                return extract_from(slab_v, e, t)
            return lax.fori_loop(begs_s[ch], begs_s[ch + 1], extract, t)
        t_total = lax.fori_loop(0, nfull, slab_loop, 0)

        pltpu.sync_copy(tailt_hbm, tbuf_v)

        def tail_extract(e, t):
            return extract_from(tbuf_v, e, t)
        t_total = lax.fori_loop(begs_s[nfull], begs_s[nfull + 1],
                                tail_extract, t_total)

        def drain(i, _):
            pltpu.make_async_copy(
                rel_out.at[pl.ds(0, HALF)],
                stage_v.at[pl.ds(0, HALF)], osem).wait()
            return 0
        lax.fori_loop(0, jnp.minimum(t_total, 8), drain, 0)


    return k(h_idx, r_idx, t_idx, ent_embed, rel_t, tail_t)


def _tc_math_body(head_ref, tail_ref, rel_ref, out_ref):
    head = head_ref[...]
    tail = tail_ref[...]
    phase = rel_ref[...] * PHASE_SCALE
    re_r = jnp.cos(phase)
    im_r = jnp.sin(phase)
    re_h = head[:, :HALF]
    im_h = head[:, HALF:]
    re_s = re_h * re_r - im_h * im_r - tail[:, :HALF]
    im_s = re_h * im_r + im_h * re_r - tail[:, HALF:]
    s = jnp.sqrt(re_s * re_s + im_s * im_s)
    out_ref[...] = MARGIN - jnp.sum(s, axis=1, keepdims=True)


def _tc_math(head, tail, rel):
    blk = 2048
    grid = (BATCH // blk,)
    return pl.pallas_call(
        _tc_math_body,
        grid=grid,
        in_specs=[
            pl.BlockSpec((blk, IN_DIM), lambda i: (i, 0)),
            pl.BlockSpec((blk, IN_DIM), lambda i: (i, 0)),
            pl.BlockSpec((blk, HALF), lambda i: (i, 0)),
        ],
        out_specs=pl.BlockSpec((blk, 1), lambda i: (i, 0)),
        out_shape=jax.ShapeDtypeStruct((BATCH, 1), jnp.float32),
    )(head, tail, rel)


def kernel(sample, ent_embed, rel_embed):
    h_idx = sample[:, 0]
    r_idx = sample[:, 1]
    t_idx = sample[:, 2]
    rel_t = rel_embed.T
    tail_t = rel_t[:, (NSLAB_TOT - 1) * SLAB:]
    head, tail, rel = _sc_gather(h_idx, r_idx, t_idx, ent_embed, rel_t, tail_t)
    return _tc_math(head, tail, rel.reshape(BATCH, HALF))


# speedup vs baseline: 2.1034x; 1.0874x over previous
import functools

import jax
import jax.numpy as jnp
from jax import lax
from jax.experimental import pallas as pl
from jax.experimental.pallas import tpu as pltpu
from jax.experimental.pallas import tpu_sc as plsc

N_ENT = 1000000
N_REL = 1000000
IN_DIM = 128
HALF = IN_DIM // 2
BATCH = 16384
MARGIN = 6.0
EPSILON = 2.0
EMB_RANGE = (MARGIN + EPSILON) / IN_DIM
PI = 3.141592653589793
PHASE_SCALE = PI / EMB_RANGE

NC = 2
NS = 16
NW = NC * NS
B_PER_W = BATCH // NW
CHUNK = 64
N_CHUNKS = B_PER_W // CHUNK

SLAB = 256
NSLAB_TOT = (N_REL + SLAB - 1) // SLAB
SLAB_PW = 123
LISTCAP = BATCH + 16


def _sc_gather(h_idx, r_idx, t_idx, ent_embed, rel_t, tail_t):
    mesh = plsc.VectorSubcoreMesh(core_axis_name="c", subcore_axis_name="s")

    @functools.partial(
        pl.kernel,
        mesh=mesh,
        compiler_params=pltpu.CompilerParams(needs_layout_passes=False),
        out_type=(
            jax.ShapeDtypeStruct((BATCH, IN_DIM), jnp.float32),
            jax.ShapeDtypeStruct((BATCH, IN_DIM), jnp.float32),
            jax.ShapeDtypeStruct((BATCH * HALF,), jnp.float32),
        ),
        scratch_types=(
            pltpu.VMEM((CHUNK,), jnp.int32),
            pltpu.VMEM((CHUNK,), jnp.int32),
            pltpu.VMEM((CHUNK, IN_DIM), jnp.float32),
            pltpu.VMEM((CHUNK, IN_DIM), jnp.float32),
            pltpu.VMEM((BATCH,), jnp.int32),
            pltpu.VMEM((LISTCAP,), jnp.int32),
            pltpu.VMEM((LISTCAP,), jnp.int32),
            pltpu.VMEM((LISTCAP,), jnp.int32),
            pltpu.VMEM((HALF, 2 * SLAB), jnp.float32),
            pltpu.VMEM((HALF, 64), jnp.float32),
            pltpu.VMEM((8 * HALF,), jnp.float32),
            pltpu.SMEM((SLAB_PW + 8,), jnp.int32),
            pltpu.SMEM((SLAB_PW + 8,), jnp.int32),
            pltpu.SMEM((SLAB_PW + 8,), jnp.int32),
            pltpu.SemaphoreType.DMA,
            pltpu.SemaphoreType.DMA,
            pltpu.SemaphoreType.DMA,
        ),
    )
    def k(h_idx_hbm, r_idx_hbm, t_idx_hbm, ent_hbm, relt_hbm, tailt_hbm,
          head_out, tail_out, rel_out,
          hidx_v, tidx_v, head_v, tail_v,
          ridx_all, lpack, lslab, lsort, slab_v, tbuf_v, stage_v,
          cnts_s, begs_s, curs_s, sem, osem, dsem):
        wid = lax.axis_index("s") * NC + lax.axis_index("c")
        base = wid * B_PER_W

        for c in range(N_CHUNKS):
            off = base + c * CHUNK
            pltpu.sync_copy(h_idx_hbm.at[pl.ds(off, CHUNK)], hidx_v)
            pltpu.sync_copy(t_idx_hbm.at[pl.ds(off, CHUNK)], tidx_v)
            cp_h = pltpu.async_copy(ent_hbm.at[hidx_v], head_v, sem)
            cp_t = pltpu.async_copy(ent_hbm.at[tidx_v], tail_v, sem)
            cp_h.wait()
            cp_t.wait()
            pltpu.sync_copy(head_v, head_out.at[pl.ds(off, CHUNK)])
            pltpu.sync_copy(tail_v, tail_out.at[pl.ds(off, CHUNK)])

        lo = wid * (SLAB_PW * SLAB)
        hi = jnp.minimum(lo + SLAB_PW * SLAB, N_REL)
        pltpu.sync_copy(r_idx_hbm, ridx_all)
        iota = lax.iota(jnp.int32, 16)
        lane0 = iota == 0

        def compress(g, cnt):
            v = ridx_all[pl.ds(g * 16, 16)]
            lo16 = jnp.full((16,), lo, jnp.int32)
            hi16 = jnp.full((16,), hi, jnp.int32)
            m = (v >= lo16) & (v < hi16)
            cs = plsc.cumsum(m.astype(jnp.int32))
            pos = jnp.full((16,), cnt, jnp.int32) + cs - 1
            rel_off = v - lo16
            b16 = iota + jnp.full((16,), g * 16, jnp.int32)
            pack = (b16 << 8) | (rel_off & (SLAB - 1))
            plsc.store_scatter(lpack, [pos], pack, mask=m)
            plsc.store_scatter(lslab, [pos], rel_off >> 8, mask=m)
            return cnt + cs[15]
        cnt = lax.fori_loop(0, BATCH // 16, compress, 0)

        def zero(i, _):
            cnts_s[i] = 0
            return 0
        lax.fori_loop(0, SLAB_PW + 1, zero, 0)

        def count(e, _):
            c = lslab[pl.ds(e, 16)][0]
            cnts_s[c] = cnts_s[c] + 1
            return 0
        lax.fori_loop(0, cnt, count, 0)

        def prefix(i, acc):
            begs_s[i] = acc
            curs_s[i] = acc
            return acc + cnts_s[i]
        total = lax.fori_loop(0, SLAB_PW + 1, prefix, 0)
        begs_s[SLAB_PW + 1] = total
        curs_s[SLAB_PW + 1] = total

        def scat(e, _):
            c = lslab[pl.ds(e, 16)][0]
            p = lpack[pl.ds(e, 16)][0]
            pos = curs_s[c]
            curs_s[c] = pos + 1
            plsc.store_scatter(lsort, [jnp.full((16,), pos, jnp.int32)],
                               jnp.full((16,), p, jnp.int32), mask=lane0)
            return 0
        lax.fori_loop(0, cnt, scat, 0)

        def extract_from(buf_ref, e, t, cbase=0):
            p = lsort[pl.ds(e, 16)][0]
            b = p >> 8
            col = (p & (SLAB - 1)) + cbase
            slot = t % 8

            @pl.when(t >= 8)
            def _():
                pltpu.make_async_copy(
                    rel_out.at[pl.ds(0, HALF)],
                    stage_v.at[pl.ds(0, HALF)], osem).wait()
            col16 = jnp.full((16,), col, jnp.int32)
            for kk in range(4):
                vals = plsc.load_gather(buf_ref, [iota + kk * 16, col16])
                stage_v[pl.ds(slot * HALF + kk * 16, 16)] = vals
            pltpu.async_copy(stage_v.at[pl.ds(slot * HALF, HALF)],
                             rel_out.at[pl.ds(b * HALF, HALF)], osem)
            return t + 1

        hi_full = jnp.minimum(lo + SLAB_PW * SLAB, (NSLAB_TOT - 1) * SLAB)
        nfull = (hi_full - lo) // SLAB
        pltpu.async_copy(relt_hbm.at[:, pl.ds(lo, SLAB)],
                         slab_v.at[:, pl.ds(0, SLAB)], dsem)

        def slab_loop(ch, t):
            pltpu.make_async_copy(
                relt_hbm.at[:, pl.ds(lo, SLAB)],
                slab_v.at[:, pl.ds(0, SLAB)], dsem).wait()

            @pl.when(ch + 1 < nfull)
            def _():
                half = pl.multiple_of(((ch + 1) % 2) * SLAB, 128)
                pltpu.async_copy(
                    relt_hbm.at[:, pl.ds(lo + (ch + 1) * SLAB, SLAB)],
                    slab_v.at[:, pl.ds(half, SLAB)], dsem)
            cbase = (ch % 2) * SLAB

            def extract(e, t):
                return extract_from(slab_v, e, t, cbase)
            return lax.fori_loop(begs_s[ch], begs_s[ch + 1], extract, t)
        t_total = lax.fori_loop(0, nfull, slab_loop, 0)

        pltpu.sync_copy(tailt_hbm, tbuf_v)

        def tail_extract(e, t):
            return extract_from(tbuf_v, e, t)
        t_total = lax.fori_loop(begs_s[nfull], begs_s[nfull + 1],
                                tail_extract, t_total)

        def drain(i, _):
            pltpu.make_async_copy(
                rel_out.at[pl.ds(0, HALF)],
                stage_v.at[pl.ds(0, HALF)], osem).wait()
            return 0
        lax.fori_loop(0, jnp.minimum(t_total, 8), drain, 0)


    return k(h_idx, r_idx, t_idx, ent_embed, rel_t, tail_t)


def _tc_math_body(head_ref, tail_ref, rel_ref, out_ref):
    head = head_ref[...]
    tail = tail_ref[...]
    phase = rel_ref[...] * PHASE_SCALE
    re_r = jnp.cos(phase)
    im_r = jnp.sin(phase)
    re_h = head[:, :HALF]
    im_h = head[:, HALF:]
    re_s = re_h * re_r - im_h * im_r - tail[:, :HALF]
    im_s = re_h * im_r + im_h * re_r - tail[:, HALF:]
    s = jnp.sqrt(re_s * re_s + im_s * im_s)
    out_ref[...] = MARGIN - jnp.sum(s, axis=1, keepdims=True)


def _tc_math(head, tail, rel):
    blk = 2048
    grid = (BATCH // blk,)
    return pl.pallas_call(
        _tc_math_body,
        grid=grid,
        in_specs=[
            pl.BlockSpec((blk, IN_DIM), lambda i: (i, 0)),
            pl.BlockSpec((blk, IN_DIM), lambda i: (i, 0)),
            pl.BlockSpec((blk, HALF), lambda i: (i, 0)),
        ],
        out_specs=pl.BlockSpec((blk, 1), lambda i: (i, 0)),
        out_shape=jax.ShapeDtypeStruct((BATCH, 1), jnp.float32),
    )(head, tail, rel)


def kernel(sample, ent_embed, rel_embed):
    h_idx = sample[:, 0]
    r_idx = sample[:, 1]
    t_idx = sample[:, 2]
    rel_t = rel_embed.T
    tail_t = rel_t[:, (NSLAB_TOT - 1) * SLAB:]
    head, tail, rel = _sc_gather(h_idx, r_idx, t_idx, ent_embed, rel_t, tail_t)
    return _tc_math(head, tail, rel.reshape(BATCH, HALF))


# speedup vs baseline: 2.1467x; 1.0206x over previous
import functools

import jax
import jax.numpy as jnp
from jax import lax
from jax.experimental import pallas as pl
from jax.experimental.pallas import tpu as pltpu
from jax.experimental.pallas import tpu_sc as plsc

N_ENT = 1000000
N_REL = 1000000
IN_DIM = 128
HALF = IN_DIM // 2
BATCH = 16384
MARGIN = 6.0
EPSILON = 2.0
EMB_RANGE = (MARGIN + EPSILON) / IN_DIM
PI = 3.141592653589793
PHASE_SCALE = PI / EMB_RANGE

NC = 2
NS = 16
NW = NC * NS
B_PER_W = BATCH // NW
CHUNK = 16
N_CHUNKS = B_PER_W // CHUNK

SLAB = 512
NSLAB_TOT = (N_REL + SLAB - 1) // SLAB
SLAB_PW = 62
LISTCAP = BATCH + 16


def _sc_gather(h_idx, r_idx, t_idx, ent_embed, rel_t, tail_t):
    mesh = plsc.VectorSubcoreMesh(core_axis_name="c", subcore_axis_name="s")

    @functools.partial(
        pl.kernel,
        mesh=mesh,
        compiler_params=pltpu.CompilerParams(needs_layout_passes=False),
        out_type=(
            jax.ShapeDtypeStruct((BATCH, IN_DIM), jnp.float32),
            jax.ShapeDtypeStruct((BATCH, IN_DIM), jnp.float32),
            jax.ShapeDtypeStruct((BATCH * HALF,), jnp.float32),
        ),
        scratch_types=(
            pltpu.VMEM((CHUNK,), jnp.int32),
            pltpu.VMEM((CHUNK,), jnp.int32),
            pltpu.VMEM((CHUNK, IN_DIM), jnp.float32),
            pltpu.VMEM((CHUNK, IN_DIM), jnp.float32),
            pltpu.VMEM((BATCH,), jnp.int32),
            pltpu.VMEM((LISTCAP,), jnp.int32),
            pltpu.VMEM((LISTCAP,), jnp.int32),
            pltpu.VMEM((HALF, 2 * SLAB), jnp.float32),
            pltpu.VMEM((HALF, 64), jnp.float32),
            pltpu.VMEM((8 * HALF,), jnp.float32),
            pltpu.SMEM((SLAB_PW + 8,), jnp.int32),
            pltpu.SMEM((SLAB_PW + 8,), jnp.int32),
            pltpu.SMEM((SLAB_PW + 8,), jnp.int32),
            pltpu.SemaphoreType.DMA,
            pltpu.SemaphoreType.DMA,
            pltpu.SemaphoreType.DMA,
        ),
    )
    def k(h_idx_hbm, r_idx_hbm, t_idx_hbm, ent_hbm, relt_hbm, tailt_hbm,
          head_out, tail_out, rel_out,
          hidx_v, tidx_v, head_v, tail_v,
          ridx_all, lpack, lsort, slab_v, tbuf_v, stage_v,
          cnts_s, begs_s, curs_s, sem, osem, dsem):
        wid = lax.axis_index("s") * NC + lax.axis_index("c")
        base = wid * B_PER_W

        for c in range(N_CHUNKS):
            off = base + c * CHUNK
            pltpu.sync_copy(h_idx_hbm.at[pl.ds(off, CHUNK)], hidx_v)
            pltpu.sync_copy(t_idx_hbm.at[pl.ds(off, CHUNK)], tidx_v)
            cp_h = pltpu.async_copy(ent_hbm.at[hidx_v], head_v, sem)
            cp_t = pltpu.async_copy(ent_hbm.at[tidx_v], tail_v, sem)
            cp_h.wait()
            cp_t.wait()
            pltpu.sync_copy(head_v, head_out.at[pl.ds(off, CHUNK)])
            pltpu.sync_copy(tail_v, tail_out.at[pl.ds(off, CHUNK)])

        lo = wid * (SLAB_PW * SLAB)
        hi = jnp.minimum(lo + SLAB_PW * SLAB, N_REL)
        pltpu.sync_copy(r_idx_hbm, ridx_all)
        iota = lax.iota(jnp.int32, 16)
        lane0 = iota == 0

        def compress(g, cnt):
            v = ridx_all[pl.ds(g * 16, 16)]
            lo16 = jnp.full((16,), lo, jnp.int32)
            hi16 = jnp.full((16,), hi, jnp.int32)
            m = (v >= lo16) & (v < hi16)
            cs = plsc.cumsum(m.astype(jnp.int32))
            pos = jnp.full((16,), cnt, jnp.int32) + cs - 1
            rel_off = v - lo16
            b16 = iota + jnp.full((16,), g * 16, jnp.int32)
            slab16 = rel_off >> 9
            pack = (slab16 << 23) | (b16 << 9) | (rel_off & (SLAB - 1))
            plsc.store_scatter(lpack, [pos], pack, mask=m)
            return cnt + cs[15]
        cnt = lax.fori_loop(0, BATCH // 16, compress, 0)

        def zero(i, _):
            cnts_s[i] = 0
            return 0
        lax.fori_loop(0, SLAB_PW + 1, zero, 0)

        def count(e, _):
            c = lpack[pl.ds(e, 16)][0] >> 23
            cnts_s[c] = cnts_s[c] + 1
            return 0
        lax.fori_loop(0, cnt, count, 0)

        def prefix(i, acc):
            begs_s[i] = acc
            curs_s[i] = acc
            return acc + cnts_s[i]
        total = lax.fori_loop(0, SLAB_PW + 1, prefix, 0)
        begs_s[SLAB_PW + 1] = total
        curs_s[SLAB_PW + 1] = total

        def scat(e, _):
            p = lpack[pl.ds(e, 16)][0]
            c = p >> 23
            pos = curs_s[c]
            curs_s[c] = pos + 1
            plsc.store_scatter(lsort, [jnp.full((16,), pos, jnp.int32)],
                               jnp.full((16,), p, jnp.int32), mask=lane0)
            return 0
        lax.fori_loop(0, cnt, scat, 0)

        def extract_from(buf_ref, e, t, cbase=0):
            p = lsort[pl.ds(e, 16)][0]
            b = (p >> 9) & (BATCH - 1)
            col = (p & (SLAB - 1)) + cbase
            slot = t % 8

            @pl.when(t >= 8)
            def _():
                pltpu.make_async_copy(
                    rel_out.at[pl.ds(0, HALF)],
                    stage_v.at[pl.ds(0, HALF)], osem).wait()
            col16 = jnp.full((16,), col, jnp.int32)
            for kk in range(4):
                vals = plsc.load_gather(buf_ref, [iota + kk * 16, col16])
                stage_v[pl.ds(slot * HALF + kk * 16, 16)] = vals
            pltpu.async_copy(stage_v.at[pl.ds(slot * HALF, HALF)],
                             rel_out.at[pl.ds(b * HALF, HALF)], osem)
            return t + 1

        hi_full = jnp.minimum(lo + SLAB_PW * SLAB, (NSLAB_TOT - 1) * SLAB)
        nfull = (hi_full - lo) // SLAB
        pltpu.async_copy(relt_hbm.at[:, pl.ds(lo, SLAB)],
                         slab_v.at[:, pl.ds(0, SLAB)], dsem)

        def slab_loop(ch, t):
            pltpu.make_async_copy(
                relt_hbm.at[:, pl.ds(lo, SLAB)],
                slab_v.at[:, pl.ds(0, SLAB)], dsem).wait()

            @pl.when(ch + 1 < nfull)
            def _():
                half = pl.multiple_of(((ch + 1) % 2) * SLAB, 128)
                pltpu.async_copy(
                    relt_hbm.at[:, pl.ds(lo + (ch + 1) * SLAB, SLAB)],
                    slab_v.at[:, pl.ds(half, SLAB)], dsem)
            cbase = (ch % 2) * SLAB

            def extract(e, t):
                return extract_from(slab_v, e, t, cbase)
            return lax.fori_loop(begs_s[ch], begs_s[ch + 1], extract, t)
        t_total = lax.fori_loop(0, nfull, slab_loop, 0)

        pltpu.sync_copy(tailt_hbm, tbuf_v)

        def tail_extract(e, t):
            return extract_from(tbuf_v, e, t)
        t_total = lax.fori_loop(begs_s[nfull], begs_s[nfull + 1],
                                tail_extract, t_total)

        def drain(i, _):
            pltpu.make_async_copy(
                rel_out.at[pl.ds(0, HALF)],
                stage_v.at[pl.ds(0, HALF)], osem).wait()
            return 0
        lax.fori_loop(0, jnp.minimum(t_total, 8), drain, 0)


    return k(h_idx, r_idx, t_idx, ent_embed, rel_t, tail_t)


def _tc_math_body(head_ref, tail_ref, rel_ref, out_ref):
    head = head_ref[...]
    tail = tail_ref[...]
    phase = rel_ref[...] * PHASE_SCALE
    re_r = jnp.cos(phase)
    im_r = jnp.sin(phase)
    re_h = head[:, :HALF]
    im_h = head[:, HALF:]
    re_s = re_h * re_r - im_h * im_r - tail[:, :HALF]
    im_s = re_h * im_r + im_h * re_r - tail[:, HALF:]
    s = jnp.sqrt(re_s * re_s + im_s * im_s)
    out_ref[...] = MARGIN - jnp.sum(s, axis=1, keepdims=True)


def _tc_math(head, tail, rel):
    blk = 2048
    grid = (BATCH // blk,)
    return pl.pallas_call(
        _tc_math_body,
        grid=grid,
        in_specs=[
            pl.BlockSpec((blk, IN_DIM), lambda i: (i, 0)),
            pl.BlockSpec((blk, IN_DIM), lambda i: (i, 0)),
            pl.BlockSpec((blk, HALF), lambda i: (i, 0)),
        ],
        out_specs=pl.BlockSpec((blk, 1), lambda i: (i, 0)),
        out_shape=jax.ShapeDtypeStruct((BATCH, 1), jnp.float32),
    )(head, tail, rel)


def kernel(sample, ent_embed, rel_embed):
    h_idx = sample[:, 0]
    r_idx = sample[:, 1]
    t_idx = sample[:, 2]
    rel_t = rel_embed.T
    tail_t = rel_t[:, (NSLAB_TOT - 1) * SLAB:]
    head, tail, rel = _sc_gather(h_idx, r_idx, t_idx, ent_embed, rel_t, tail_t)
    return _tc_math(head, tail, rel.reshape(BATCH, HALF))
